# baseline (speedup 1.0000x reference)
import functools

import jax
import jax.numpy as jnp
from jax import lax
from jax.experimental import pallas as pl
from jax.experimental.pallas import tpu as pltpu
from jax.experimental.pallas import tpu_sc as plsc

NUM_NODES = 100000
EMBED_DIM = 128

SPLIT = 64000
TC_BLOCK = 16000

NUM_CORES = 2
NUM_SUBCORES = 16
NUM_WORKERS = NUM_CORES * NUM_SUBCORES
CHUNK_ROWS = 400
SC_ROWS = NUM_NODES - SPLIT
NUM_CHUNKS = SC_ROWS // CHUNK_ROWS
MAX_K = -(-NUM_CHUNKS // NUM_WORKERS)
NBUF = 2


def _tc_body(in_ref, out_ref):
    out_ref[...] = in_ref[...]


def _tc_copy(table):
    return pl.pallas_call(
        _tc_body,
        grid=(SPLIT // TC_BLOCK,),
        in_specs=[pl.BlockSpec((TC_BLOCK, EMBED_DIM), lambda i: (i, 0))],
        out_specs=pl.BlockSpec((TC_BLOCK, EMBED_DIM), lambda i: (i, 0)),
        out_shape=jax.ShapeDtypeStruct((SPLIT, EMBED_DIM), table.dtype),
    )(table)


def _sc_copy(table):
    mesh = plsc.VectorSubcoreMesh(core_axis_name="c", subcore_axis_name="s")

    @functools.partial(
        pl.kernel,
        mesh=mesh,
        out_type=jax.ShapeDtypeStruct((SC_ROWS, EMBED_DIM), table.dtype),
        scratch_types=[
            pltpu.VMEM((NBUF, CHUNK_ROWS, EMBED_DIM), jnp.float32),
            pltpu.SemaphoreType.DMA((NBUF,)),
            pltpu.SemaphoreType.DMA((NBUF,)),
        ],
    )
    def copy_k(table_hbm, out_hbm, bufs, in_sems, out_sems):
        wid = lax.axis_index("s") * NUM_CORES + lax.axis_index("c")

        def in_dma(k, slot):
            c = wid + k * NUM_WORKERS
            return pltpu.make_async_copy(
                table_hbm.at[pl.ds(SPLIT + c * CHUNK_ROWS, CHUNK_ROWS)],
                bufs.at[slot],
                in_sems.at[slot],
            )

        def out_dma(k, slot):
            c = wid + k * NUM_WORKERS
            return pltpu.make_async_copy(
                bufs.at[slot],
                out_hbm.at[pl.ds(c * CHUNK_ROWS, CHUNK_ROWS)],
                out_sems.at[slot],
            )

        def valid(k):
            return wid + k * NUM_WORKERS < NUM_CHUNKS

        for k in range(min(NBUF - 1, MAX_K)):
            @pl.when(valid(k))
            def _(k=k):
                in_dma(k, k % NBUF).start()

        for k in range(MAX_K):
            slot = k % NBUF
            kp = k + NBUF - 1
            if kp < MAX_K:
                @pl.when(valid(kp))
                def _(kp=kp):
                    prev = kp - NBUF
                    if prev >= 0:
                        out_dma(prev, kp % NBUF).wait()
                    in_dma(kp, kp % NBUF).start()

            @pl.when(valid(k))
            def _(k=k, slot=slot):
                in_dma(k, slot).wait()
                out_dma(k, slot).start()

        for k in range(MAX_K):
            @pl.when(jnp.logical_and(valid(k), jnp.logical_not(valid(k + NBUF))))
            def _(k=k):
                out_dma(k, k % NBUF).wait()

    return copy_k(table)


def kernel(embedding_table):
    tc_part = _tc_copy(embedding_table)
    sc_part = _sc_copy(embedding_table)
    return jnp.concatenate([tc_part, sc_part], axis=0)


# speedup vs baseline: 1.4865x; 1.4865x over previous
import functools

import jax
import jax.numpy as jnp
from jax import lax
from jax.experimental import pallas as pl
from jax.experimental.pallas import tpu as pltpu
from jax.experimental.pallas import tpu_sc as plsc

NUM_NODES = 100000
EMBED_DIM = 128
NUM_CORES = 2
NUM_SUBCORES = 16
NUM_WORKERS = NUM_CORES * NUM_SUBCORES
CHUNK_ROWS = 400
NUM_CHUNKS = NUM_NODES // CHUNK_ROWS
MAX_K = -(-NUM_CHUNKS // NUM_WORKERS)
NBUF = 2


def kernel(embedding_table):
    n, d = embedding_table.shape
    mesh = plsc.VectorSubcoreMesh(core_axis_name="c", subcore_axis_name="s")

    @functools.partial(
        pl.kernel,
        mesh=mesh,
        out_type=jax.ShapeDtypeStruct((n, d), embedding_table.dtype),
        scratch_types=[
            pltpu.VMEM((NBUF, CHUNK_ROWS, EMBED_DIM), jnp.float32),
            pltpu.SemaphoreType.DMA((NBUF,)),
            pltpu.SemaphoreType.DMA((NBUF,)),
        ],
    )
    def copy_k(table_hbm, out_hbm, bufs, in_sems, out_sems):
        wid = lax.axis_index("s") * NUM_CORES + lax.axis_index("c")

        def in_dma(k, slot):
            c = wid + k * NUM_WORKERS
            return pltpu.make_async_copy(
                table_hbm.at[pl.ds(c * CHUNK_ROWS, CHUNK_ROWS)],
                bufs.at[slot],
                in_sems.at[slot],
            )

        def out_dma(k, slot):
            c = wid + k * NUM_WORKERS
            return pltpu.make_async_copy(
                bufs.at[slot],
                out_hbm.at[pl.ds(c * CHUNK_ROWS, CHUNK_ROWS)],
                out_sems.at[slot],
            )

        def valid(k):
            return wid + k * NUM_WORKERS < NUM_CHUNKS

        for k in range(min(NBUF - 1, MAX_K)):
            @pl.when(valid(k))
            def _(k=k):
                in_dma(k, k % NBUF).start()

        for k in range(MAX_K):
            slot = k % NBUF
            kp = k + NBUF - 1
            if kp < MAX_K:
                @pl.when(valid(kp))
                def _(kp=kp):
                    prev = kp - NBUF
                    if prev >= 0:
                        out_dma(prev, kp % NBUF).wait()
                    in_dma(kp, kp % NBUF).start()

            @pl.when(valid(k))
            def _(k=k, slot=slot):
                in_dma(k, slot).wait()
                out_dma(k, slot).start()

        for k in range(MAX_K):
            @pl.when(jnp.logical_and(valid(k), jnp.logical_not(valid(k + NBUF))))
            def _(k=k):
                out_dma(k, k % NBUF).wait()

    return copy_k(embedding_table)
